# baseline (speedup 1.0000x reference)
import jax
import jax.numpy as jnp
from jax import lax
from jax.experimental import pallas as pl
from jax.experimental.pallas import tpu as pltpu
from jax.experimental.pallas import tpu_sc as plsc

NUM_SEG = 10000
BATCH = 4
N_ROWS = 160000
D = 128
NC = 2
NS = 16
ROWS_PER_TILE = N_ROWS // NS
CHUNK = 80
NCHUNK = ROWS_PER_TILE // CHUNK
NBUF = 4
HALF = 64
SEG_PER_TILE = 624
SEG_TAIL = NUM_SEG - NS * SEG_PER_TILE
ROUNDS = BATCH // NC


def _copy_acc_slice(s, src, dst):
    seg0 = s * SEG_PER_TILE
    pltpu.sync_copy(src.at[pl.ds(seg0, SEG_PER_TILE)],
                    dst.at[pl.ds(seg0, SEG_PER_TILE)])

    @pl.when(s == NS - 1)
    def _():
        t0 = NS * SEG_PER_TILE
        pltpu.sync_copy(src.at[pl.ds(t0, SEG_TAIL)], dst.at[pl.ds(t0, SEG_TAIL)])


def _seg_sum_body(data_hbm, ids_hbm, zeros_hbm, out_hbm,
                  idx_v, rows, sems, ssem, acc_sh):
    c = lax.axis_index("c")
    s = lax.axis_index("s")

    def wbase(r):
        b = c * ROUNDS + r
        return b * NS + s

    def start(j, k, base):
        @pl.when(j < NCHUNK)
        def _():
            pltpu.async_copy(
                data_hbm.at[pl.ds(base + j * CHUNK, CHUNK)], rows[k], sems[k])

    def wait(k):
        pltpu.make_async_copy(
            data_hbm.at[pl.ds(0, CHUNK)], rows[k], sems[k]).wait()

    def scat(j, k):
        pltpu.sync_copy(rows[k], acc_sh.at[idx_v.at[j]], add=True)

    def scat_async(j, k):
        return pltpu.async_copy(rows[k], acc_sh.at[idx_v.at[j]], ssem,
                                add=True)

    def load_ids(w, h):
        nid = HALF if h == 0 else NCHUNK - HALF
        pltpu.sync_copy(ids_hbm.at[w, pl.ds(h * HALF, nid)],
                        idx_v.at[pl.ds(0, nid)])

    def prologue(r):
        base = wbase(r) * ROWS_PER_TILE
        for k in range(NBUF):
            start(k, k, base)
        load_ids(wbase(r), 0)

    prologue(0)
    _copy_acc_slice(s, zeros_hbm, acc_sh)
    plsc.subcore_barrier()

    for r in range(ROUNDS):
        b = c * ROUNDS + r
        w = wbase(r)
        base = w * ROWS_PER_TILE

        for h, nquad in ((0, HALF // NBUF), (1, (NCHUNK - HALF - 1) // NBUF)):
            h0 = h * HALF
            if h > 0:
                load_ids(w, h)

            def quad_body(g, carry):
                j = h0 + NBUF * g
                for k in (0, 2):
                    wait(k)
                    wait(k + 1)
                    da = scat_async(j + k - h0, k)
                    db = scat_async(j + k + 1 - h0, k + 1)
                    da.wait()
                    db.wait()
                    start(j + k + NBUF, k, base)
                    start(j + k + NBUF + 1, k + 1, base)
                return carry

            lax.fori_loop(0, nquad, quad_body, 0)

        wait(0)
        scat(NCHUNK - 1 - HALF, 0)
        plsc.subcore_barrier()

        if r + 1 < ROUNDS:
            prologue(r + 1)
        _copy_acc_slice(s, acc_sh, out_hbm.at[pl.ds(b * NUM_SEG, NUM_SEG)])
        if r + 1 < ROUNDS:
            _copy_acc_slice(s, zeros_hbm, acc_sh)
            plsc.subcore_barrier()


def kernel(data, segment_ids):
    data2 = data.reshape(BATCH * N_ROWS, D)
    ids3 = segment_ids.astype(jnp.int32).reshape(BATCH * NS, NCHUNK, CHUNK)
    zeros = jnp.zeros((NUM_SEG, D), jnp.float32)

    f = pl.kernel(
        _seg_sum_body,
        out_type=jax.ShapeDtypeStruct((BATCH * NUM_SEG, D), jnp.float32),
        mesh=plsc.VectorSubcoreMesh(core_axis_name="c", subcore_axis_name="s"),
        scratch_types=[
            pltpu.VMEM((HALF, CHUNK), jnp.int32),
            [pltpu.VMEM((CHUNK, D), jnp.float32)] * NBUF,
            [pltpu.SemaphoreType.DMA] * NBUF,
            pltpu.SemaphoreType.DMA,
            pltpu.VMEM_SHARED((NUM_SEG, D), jnp.float32),
        ],
    )
    out = f(data2, ids3, zeros)
    return out.reshape(BATCH, NUM_SEG, D)


# speedup vs baseline: 1.0089x; 1.0089x over previous
import jax
import jax.numpy as jnp
from jax import lax
from jax.experimental import pallas as pl
from jax.experimental.pallas import tpu as pltpu
from jax.experimental.pallas import tpu_sc as plsc

NUM_SEG = 10000
BATCH = 4
N_ROWS = 160000
D = 128
NC = 2
NS = 16
ROWS_PER_TILE = N_ROWS // NS
SUB = 80
CHUNK = 2 * SUB
NCHUNK = ROWS_PER_TILE // CHUNK
NSUBT = ROWS_PER_TILE // SUB
HALF = 64
SEG_PER_TILE = 624
SEG_TAIL = NUM_SEG - NS * SEG_PER_TILE
ROUNDS = BATCH // NC


def _copy_acc_slice(s, src, dst):
    seg0 = s * SEG_PER_TILE
    pltpu.sync_copy(src.at[pl.ds(seg0, SEG_PER_TILE)],
                    dst.at[pl.ds(seg0, SEG_PER_TILE)])

    @pl.when(s == NS - 1)
    def _():
        t0 = NS * SEG_PER_TILE
        pltpu.sync_copy(src.at[pl.ds(t0, SEG_TAIL)], dst.at[pl.ds(t0, SEG_TAIL)])


def _seg_sum_body(data_hbm, ids_hbm, zeros_hbm, out_hbm,
                  idx_v, rows, sems, acc_sh):
    c = lax.axis_index("c")
    s = lax.axis_index("s")

    def wbase(r):
        b = c * ROUNDS + r
        return b * NS + s

    def start(j, k, base):
        pltpu.async_copy(
            data_hbm.at[pl.ds(base + j * CHUNK, CHUNK)], rows[k], sems[k])

    def wait(k):
        pltpu.make_async_copy(
            data_hbm.at[pl.ds(0, CHUNK)], rows[k], sems[k]).wait()

    def start_tail(k, base):
        pltpu.async_copy(
            data_hbm.at[pl.ds(base + NCHUNK * CHUNK, SUB)],
            rows[k].at[pl.ds(0, SUB)], sems[k])

    def wait_tail(k):
        pltpu.make_async_copy(
            data_hbm.at[pl.ds(0, SUB)], rows[k].at[pl.ds(0, SUB)],
            sems[k]).wait()

    def scat2(j, k, h0):
        for kk in range(2):
            pltpu.sync_copy(rows[k].at[pl.ds(kk * SUB, SUB)],
                            acc_sh.at[idx_v.at[2 * j + kk - h0]], add=True)

    def scat_tail(k):
        pltpu.sync_copy(rows[k].at[pl.ds(0, SUB)],
                        acc_sh.at[idx_v.at[NSUBT - 1 - HALF]], add=True)

    def load_ids(w, h):
        nid = HALF if h == 0 else NSUBT - HALF
        pltpu.sync_copy(ids_hbm.at[w, pl.ds(h * HALF, nid)],
                        idx_v.at[pl.ds(0, nid)])

    def prologue(r):
        base = wbase(r) * ROWS_PER_TILE
        start(0, 0, base)
        start(1, 1, base)
        load_ids(wbase(r), 0)

    prologue(0)
    _copy_acc_slice(s, zeros_hbm, acc_sh)
    plsc.subcore_barrier()

    for r in range(ROUNDS):
        b = c * ROUNDS + r
        w = wbase(r)
        base = w * ROWS_PER_TILE

        def pair_body(h0):
            def body(g, carry):
                j = 2 * g
                wait(0)
                scat2(j, 0, h0)
                start(j + 2, 0, base)
                wait(1)
                scat2(j + 1, 1, h0)
                start(j + 3, 1, base)
                return carry
            return body

        lax.fori_loop(0, 15, pair_body(0), 0)
        wait(0)
        scat2(30, 0, 0)
        start(32, 0, base)
        wait(1)
        scat2(31, 1, 0)
        start(33, 1, base)

        load_ids(w, 1)

        def body2(g, carry):
            j = 2 * g + 32
            wait(0)
            scat2(j, 0, HALF)
            @pl.when(j + 2 < NCHUNK)
            def _():
                start(j + 2, 0, base)
            wait(1)
            scat2(j + 1, 1, HALF)
            @pl.when(j + 3 < NCHUNK)
            def _():
                start(j + 3, 1, base)
            return carry

        lax.fori_loop(0, 14, body2, 0)
        wait(0)
        scat2(60, 0, HALF)
        start_tail(0, base)
        wait(1)
        scat2(61, 1, HALF)
        wait_tail(0)
        scat_tail(0)
        plsc.subcore_barrier()

        if r + 1 < ROUNDS:
            prologue(r + 1)
        _copy_acc_slice(s, acc_sh, out_hbm.at[pl.ds(b * NUM_SEG, NUM_SEG)])
        if r + 1 < ROUNDS:
            _copy_acc_slice(s, zeros_hbm, acc_sh)
            plsc.subcore_barrier()


def kernel(data, segment_ids):
    data2 = data.reshape(BATCH * N_ROWS, D)
    ids3 = segment_ids.astype(jnp.int32).reshape(BATCH * NS, NSUBT, SUB)
    zeros = jnp.zeros((NUM_SEG, D), jnp.float32)

    f = pl.kernel(
        _seg_sum_body,
        out_type=jax.ShapeDtypeStruct((BATCH * NUM_SEG, D), jnp.float32),
        mesh=plsc.VectorSubcoreMesh(core_axis_name="c", subcore_axis_name="s"),
        scratch_types=[
            pltpu.VMEM((HALF, SUB), jnp.int32),
            [pltpu.VMEM((CHUNK, D), jnp.float32)] * 2,
            [pltpu.SemaphoreType.DMA] * 2,
            pltpu.VMEM_SHARED((NUM_SEG, D), jnp.float32),
        ],
    )
    out = f(data2, ids3, zeros)
    return out.reshape(BATCH, NUM_SEG, D)


# speedup vs baseline: 1.0614x; 1.0521x over previous
import jax
import jax.numpy as jnp
from jax import lax
from jax.experimental import pallas as pl
from jax.experimental.pallas import tpu as pltpu
from jax.experimental.pallas import tpu_sc as plsc

NUM_SEG = 10000
BATCH = 4
N_ROWS = 160000
D = 128
NC = 2
NS = 16
ROWS_PER_TILE = N_ROWS // NS
CHUNK = 80
NCHUNK = ROWS_PER_TILE // CHUNK
NBUF = 4
HALF = 64
SEG_PER_TILE = 624
SEG_TAIL = NUM_SEG - NS * SEG_PER_TILE
ROUNDS = BATCH // NC


def _copy_acc_slice(s, src, dst):
    seg0 = s * SEG_PER_TILE
    pltpu.sync_copy(src.at[pl.ds(seg0, SEG_PER_TILE)],
                    dst.at[pl.ds(seg0, SEG_PER_TILE)])

    @pl.when(s == NS - 1)
    def _():
        t0 = NS * SEG_PER_TILE
        pltpu.sync_copy(src.at[pl.ds(t0, SEG_TAIL)], dst.at[pl.ds(t0, SEG_TAIL)])


def _seg_sum_body(data_hbm, ids_hbm, zeros_hbm, out_hbm,
                  idx_v, rows, sems, acc_sh):
    c = lax.axis_index("c")
    s = lax.axis_index("s")

    def wbase(r):
        b = c * ROUNDS + r
        return b * NS + s

    def start(j, k, base):
        @pl.when(j < NCHUNK)
        def _():
            pltpu.async_copy(
                data_hbm.at[pl.ds(base + j * CHUNK, CHUNK)], rows[k], sems[k])

    def wait(k):
        pltpu.make_async_copy(
            data_hbm.at[pl.ds(0, CHUNK)], rows[k], sems[k]).wait()

    def scat(j, k):
        pltpu.sync_copy(rows[k], acc_sh.at[idx_v.at[j]], add=True)

    def load_ids(w, h):
        nid = HALF if h == 0 else NCHUNK - HALF
        pltpu.sync_copy(ids_hbm.at[w, pl.ds(h * HALF, nid)],
                        idx_v.at[pl.ds(0, nid)])

    def prologue(r):
        base = wbase(r) * ROWS_PER_TILE
        for k in range(NBUF):
            start(k, k, base)
        load_ids(wbase(r), 0)

    prologue(0)
    _copy_acc_slice(s, zeros_hbm, acc_sh)
    plsc.subcore_barrier()

    for r in range(ROUNDS):
        b = c * ROUNDS + r
        w = wbase(r)
        base = w * ROWS_PER_TILE

        for h, nquad in ((0, HALF // NBUF), (1, (NCHUNK - HALF - 1) // NBUF)):
            h0 = h * HALF
            if h > 0:
                load_ids(w, h)

            def quad_body(g, carry):
                j = h0 + NBUF * g
                for k in range(NBUF):
                    wait(k)
                    scat(j + k - h0, k)
                    start(j + k + NBUF, k, base)
                return carry

            lax.fori_loop(0, nquad, quad_body, 0)

        wait(0)
        scat(NCHUNK - 1 - HALF, 0)
        plsc.subcore_barrier()

        if r + 1 < ROUNDS:
            prologue(r + 1)
        _copy_acc_slice(s, acc_sh, out_hbm.at[pl.ds(b * NUM_SEG, NUM_SEG)])
        if r + 1 < ROUNDS:
            _copy_acc_slice(s, zeros_hbm, acc_sh)
            plsc.subcore_barrier()


def kernel(data, segment_ids):
    data2 = data.reshape(BATCH * N_ROWS, D)
    ids3 = segment_ids.astype(jnp.int32).reshape(BATCH * NS, NCHUNK, CHUNK)
    zeros = jnp.zeros((NUM_SEG, D), jnp.float32)

    f = pl.kernel(
        _seg_sum_body,
        out_type=jax.ShapeDtypeStruct((BATCH * NUM_SEG, D), jnp.float32),
        mesh=plsc.VectorSubcoreMesh(core_axis_name="c", subcore_axis_name="s"),
        scratch_types=[
            pltpu.VMEM((HALF, CHUNK), jnp.int32),
            [pltpu.VMEM((CHUNK, D), jnp.float32)] * NBUF,
            [pltpu.SemaphoreType.DMA] * NBUF,
            pltpu.VMEM_SHARED((NUM_SEG, D), jnp.float32),
        ],
    )
    out = f(data2, ids3, zeros)
    return out.reshape(BATCH, NUM_SEG, D)
